# baseline (speedup 1.0000x reference)
import jax
import jax.numpy as jnp
from jax.experimental import pallas as pl
from jax.experimental.pallas import tpu as pltpu

N = 10000
NFEAT = 128
NHID = 64
NBUF = 5
M_CHUNK = 80
NCHUNKS = N // M_CHUNK
NMAIN = (NCHUNKS // NBUF) * NBUF


def _gcn_body(b_ref, x_hbm, w_hbm, adj_hbm, out_ref,
              supp_ref, x_vmem, w_vmem, xsem, wsem, *rest):
    bufs = rest[:NBUF]
    sems = rest[NBUF:]

    def start(chunk, slot):
        pltpu.make_async_copy(
            adj_hbm.at[pl.ds(chunk * M_CHUNK, M_CHUNK), :], bufs[slot], sems[slot]
        ).start()

    def wait(slot):
        pltpu.make_async_copy(
            adj_hbm.at[pl.ds(0, M_CHUNK), :], bufs[slot], sems[slot]
        ).wait()

    xcp = pltpu.make_async_copy(x_hbm, x_vmem, xsem)
    wcp = pltpu.make_async_copy(w_hbm, w_vmem, wsem)
    xcp.start()
    wcp.start()
    for s in range(NBUF):
        start(s, s)
    xcp.wait()
    wcp.wait()

    supp_ref[...] = jnp.dot(
        x_vmem[...], w_vmem[...], preferred_element_type=jnp.float32
    )
    supp = supp_ref[...].astype(jnp.bfloat16)
    bias = b_ref[...]

    def process(c, s):
        wait(s)
        acc = jnp.dot(
            bufs[s][...].astype(jnp.bfloat16), supp,
            preferred_element_type=jnp.float32,
        )
        out_ref[pl.ds(c * M_CHUNK, M_CHUNK), :] = jnp.maximum(acc + bias, 0.0)

    def outer(o, carry):
        for s in range(NBUF):
            c = o * NBUF + s
            process(c, s)

            @pl.when(c < NCHUNKS - NBUF)
            def _():
                start(c + NBUF, s)

        return carry

    jax.lax.fori_loop(0, NCHUNKS // NBUF, outer, 0)
    for s in range(NCHUNKS - NMAIN):
        process(NMAIN + s, s)


@jax.jit
def kernel(x, adj, W, b):
    n, nfeat = x.shape
    nhid = W.shape[1]
    return pl.pallas_call(
        _gcn_body,
        in_specs=[
            pl.BlockSpec((1, nhid), lambda: (0, 0)),
            pl.BlockSpec(memory_space=pl.ANY),
            pl.BlockSpec(memory_space=pl.ANY),
            pl.BlockSpec(memory_space=pl.ANY),
        ],
        out_specs=pl.BlockSpec((n, nhid), lambda: (0, 0)),
        out_shape=jax.ShapeDtypeStruct((n, nhid), jnp.float32),
        scratch_shapes=(
            [
                pltpu.VMEM((N, NHID), jnp.float32),
                pltpu.VMEM((N, NFEAT), jnp.float32),
                pltpu.VMEM((NFEAT, NHID), jnp.float32),
                pltpu.SemaphoreType.DMA,
                pltpu.SemaphoreType.DMA,
            ]
            + [pltpu.VMEM((M_CHUNK, N), jnp.float32) for _ in range(NBUF)]
            + [pltpu.SemaphoreType.DMA for _ in range(NBUF)]
        ),
    )(b.reshape(1, nhid), x, W, adj)
